# Initial kernel scaffold; baseline (speedup 1.0000x reference)
import jax
import jax.numpy as jnp
from jax.experimental import pallas as pl


def kernel(query, values, W1_w, W1_b, W2_w, W2_b, V_w, V_b):
    raise NotImplementedError("write your pallas kernel here")


# speedup vs baseline: 2.2838x; 2.2838x over previous
import jax
import jax.numpy as jnp
from jax.experimental import pallas as pl

B, N, FIN, FOUT = 128, 384, 256, 32
NR, RW = 64, 6
BB = 8


def _ring_attn_kernel(q_ref, v_ref, w1_ref, w2_ref, bias_ref, vrow_ref,
                      ctx_ref, att_ref):
    qf = q_ref[...].reshape(BB * N, FIN)
    vf = v_ref[...].reshape(BB * N, FIN)
    h = jnp.tanh(qf @ w1_ref[...] + vf @ w2_ref[...] + bias_ref[...])
    hr = h.reshape(BB * NR, RW, FOUT)
    s6 = jnp.sum(hr * vrow_ref[...].reshape(1, 1, FOUT), axis=2)
    m = jnp.max(s6, axis=1, keepdims=True)
    e = jnp.exp(s6 - m)
    d = jnp.sum(e, axis=1, keepdims=True) + e[:, :1]
    a = e / d
    col = jax.lax.broadcasted_iota(jnp.int32, (BB * NR, RW), 1)
    wgt = jnp.where(col == 0, a * 2.0, a)
    vr = vf.reshape(BB * NR, RW, FIN)
    ctx = jnp.sum(vr * wgt[:, :, None], axis=1)
    ctx_ref[...] = ctx.reshape(BB, NR, FIN)
    a7 = jnp.concatenate([a, a[:, :1]], axis=1)
    att_ref[...] = a7.reshape(BB, NR, RW + 1)


def kernel(query, values, W1_w, W1_b, W2_w, W2_b, V_w, V_b):
    bias = (W1_b + W2_b).reshape(1, FOUT)
    vrow = V_w.reshape(1, FOUT)
    grid = (B // BB,)
    ctx, att = pl.pallas_call(
        _ring_attn_kernel,
        grid=grid,
        in_specs=[
            pl.BlockSpec((BB, N, FIN), lambda i: (i, 0, 0)),
            pl.BlockSpec((BB, N, FIN), lambda i: (i, 0, 0)),
            pl.BlockSpec((FIN, FOUT), lambda i: (0, 0)),
            pl.BlockSpec((FIN, FOUT), lambda i: (0, 0)),
            pl.BlockSpec((1, FOUT), lambda i: (0, 0)),
            pl.BlockSpec((1, FOUT), lambda i: (0, 0)),
        ],
        out_specs=[
            pl.BlockSpec((BB, NR, FIN), lambda i: (i, 0, 0)),
            pl.BlockSpec((BB, NR, RW + 1), lambda i: (i, 0, 0)),
        ],
        out_shape=[
            jax.ShapeDtypeStruct((B, NR, FIN), jnp.float32),
            jax.ShapeDtypeStruct((B, NR, RW + 1), jnp.float32),
        ],
    )(query, values, W1_w, W2_w, bias, vrow)
    return ctx, att[..., None]
